# baseline (speedup 1.0000x reference)
import jax
import jax.numpy as jnp
from jax.experimental import pallas as pl
from jax.experimental.pallas import tpu as pltpu

N = 10000
D_FEAT = 128
D_EDGE = 16
D_OUT = 128


def _final_body(agg_ref, w_ref, b_ref, nd_ref, o_ref):
    acc = jnp.dot(agg_ref[...], w_ref[...], preferred_element_type=jnp.float32)
    o_ref[...] = acc * nd_ref[...] + b_ref[...]


def kernel(feat, edge_index, edge_feat, weight, bias):
    E = edge_index.shape[1]
    src = edge_index[0]
    dst = edge_index[1]
    ones = jnp.ones((E,), dtype=jnp.float32)
    out_deg = jax.ops.segment_sum(ones, src, num_segments=N)
    norm_src = jax.lax.rsqrt(jnp.clip(out_deg, 1.0, None))
    feat_n = feat * norm_src[:, None]
    m = jnp.concatenate([feat_n[src], edge_feat], axis=1)
    agg = jax.ops.segment_sum(m, dst, num_segments=N)
    in_deg = jax.ops.segment_sum(ones, dst, num_segments=N)
    norm_dst = jax.lax.rsqrt(jnp.clip(in_deg, 1.0, None))

    BLK = 400
    rst = pl.pallas_call(
        _final_body,
        grid=(N // BLK,),
        in_specs=[
            pl.BlockSpec((BLK, D_FEAT + D_EDGE), lambda i: (i, 0)),
            pl.BlockSpec((D_FEAT + D_EDGE, D_OUT), lambda i: (0, 0)),
            pl.BlockSpec((1, D_OUT), lambda i: (0, 0)),
            pl.BlockSpec((BLK, 1), lambda i: (i, 0)),
        ],
        out_specs=pl.BlockSpec((BLK, D_OUT), lambda i: (i, 0)),
        out_shape=jax.ShapeDtypeStruct((N, D_OUT), jnp.float32),
    )(agg, weight, bias[None, :], norm_dst[:, None])
    return rst


# speedup vs baseline: 5.4143x; 5.4143x over previous
import functools

import jax
import jax.numpy as jnp
from jax import lax
from jax.experimental import pallas as pl
from jax.experimental.pallas import tpu as pltpu
from jax.experimental.pallas import tpu_sc as plsc

N = 10000
D_FEAT = 128
D_EDGE = 16
D_OUT = 128
CH = 128
NP = N + 112
RPT = NP // 16
NSC = 2
NSUB = 16
NW = NSC * NSUB

_MESH = plsc.VectorSubcoreMesh(core_axis_name="c", subcore_axis_name="s")
_SC_PARAMS = pltpu.CompilerParams(use_tc_tiling_on_sc=False)


def _zero_vmem_2d(buf, rows):
    cols = buf.shape[1]

    def body(i, carry):
        for cc in range(cols // 16):
            buf[i, pl.ds(cc * 16, 16)] = jnp.zeros((16,), jnp.float32)
        return carry

    lax.fori_loop(0, rows, body, 0)


def _zero_shared_slice(zbuf, shared, row0):
    for j in range(RPT // 64):
        pltpu.sync_copy(zbuf, shared.at[pl.ds(row0 + j * 64, 64), :])
    rem = RPT % 64
    if rem:
        pltpu.sync_copy(
            zbuf.at[pl.ds(0, rem), :],
            shared.at[pl.ds(row0 + (RPT // 64) * 64, rem), :],
        )


def _drain_shared_slice(shared, stage, out_ref, cid, row0):
    nfull, rem = RPT // CH, RPT % CH
    for j in range(nfull):
        pltpu.sync_copy(shared.at[pl.ds(row0 + j * CH, CH), :], stage)
        pltpu.sync_copy(stage, out_ref.at[cid, pl.ds(row0 + j * CH, CH), :])
    if rem:
        pltpu.sync_copy(
            shared.at[pl.ds(row0 + nfull * CH, rem), :],
            stage.at[pl.ds(0, rem), :],
        )
        pltpu.sync_copy(
            stage.at[pl.ds(0, rem), :],
            out_ref.at[cid, pl.ds(row0 + nfull * CH, rem), :],
        )


def _deg_body(edge_ref, out_ref, deg_sh, ones_v, idx_v, zbuf, stage):
    cid = lax.axis_index("c")
    sid = lax.axis_index("s")
    nedge = edge_ref.shape[0] // 2
    nrow = nedge // CH

    _zero_vmem_2d(zbuf, 64)

    def ones_body(i, carry):
        ones_v[i, :] = jnp.ones((16,), jnp.float32)
        return carry

    lax.fori_loop(0, CH, ones_body, 0)

    row0 = sid * RPT
    _zero_shared_slice(zbuf, deg_sh, row0)
    plsc.subcore_barrier()

    nch = jnp.where(sid < nrow % NSUB, nrow // NSUB + 1, nrow // NSUB)

    def body(k, carry):
        row = sid + NSUB * k
        pltpu.sync_copy(edge_ref.at[pl.ds(cid * nedge + row * CH, CH)], idx_v)
        pltpu.sync_copy(ones_v, deg_sh.at[idx_v], add=True)
        return carry

    lax.fori_loop(0, nch, body, 0)
    plsc.subcore_barrier()

    pltpu.sync_copy(deg_sh.at[pl.ds(row0, RPT), :], stage)
    pltpu.sync_copy(stage, out_ref.at[cid, pl.ds(row0, RPT), :])


@jax.jit
def _deg_sc(edge_flat):
    return pl.kernel(
        _deg_body,
        out_type=jax.ShapeDtypeStruct((NSC, NP, 16), jnp.float32),
        mesh=_MESH,
        scratch_types=[
            pltpu.VMEM_SHARED((NP, 16), jnp.float32),
            pltpu.VMEM((CH, 16), jnp.float32),
            pltpu.VMEM((CH,), jnp.int32),
            pltpu.VMEM((64, 16), jnp.float32),
            pltpu.VMEM((RPT, 16), jnp.float32),
        ],
        compiler_params=_SC_PARAMS,
    )(edge_flat)


def _h_body(deg_ref, feat_ref, wf_ref, h_ref):
    d = deg_ref[0, :, 0:1]
    nd = lax.rsqrt(jnp.maximum(d, 1.0))
    h_ref[...] = jnp.dot(
        feat_ref[...] * nd, wf_ref[...], preferred_element_type=jnp.float32
    )


@jax.jit
def _h_tc(deg, feat, wf):
    blk = 1000
    return pl.pallas_call(
        _h_body,
        grid=(N // blk,),
        in_specs=[
            pl.BlockSpec((1, blk, 16), lambda i: (0, i, 0)),
            pl.BlockSpec((blk, D_FEAT), lambda i: (i, 0)),
            pl.BlockSpec((D_FEAT, D_OUT), lambda i: (0, 0)),
        ],
        out_specs=pl.BlockSpec((blk, D_OUT), lambda i: (i, 0)),
        out_shape=jax.ShapeDtypeStruct((N, D_OUT), jnp.float32),
    )(deg, feat, wf)


def _agg_body(h_ref, edge_ref, ef_ref, aggh_out, agge_out,
              aggh_sh, agge_sh, idxs_v, idxd_v, rows_v, erows_v,
              zbuf, zbufe, sem):
    cid = lax.axis_index("c")
    sid = lax.axis_index("s")
    wid = cid * NSUB + sid
    nedge = edge_ref.shape[0] // 2
    nrow = nedge // CH

    _zero_vmem_2d(zbuf, 64)
    _zero_vmem_2d(zbufe, 64)

    row0 = sid * RPT
    _zero_shared_slice(zbuf, aggh_sh, row0)
    _zero_shared_slice(zbufe, agge_sh, row0)
    plsc.subcore_barrier()

    nch = jnp.where(wid < nrow % NW, nrow // NW + 1, nrow // NW)

    def body(k, carry):
        row = wid + NW * k
        pltpu.sync_copy(edge_ref.at[pl.ds(row * CH, CH)], idxs_v)
        pltpu.sync_copy(edge_ref.at[pl.ds(nedge + row * CH, CH)], idxd_v)
        pltpu.async_copy(h_ref.at[idxs_v], rows_v, sem).wait()
        pltpu.sync_copy(ef_ref.at[pl.ds(row * CH, CH), :], erows_v)
        pltpu.sync_copy(rows_v, aggh_sh.at[idxd_v], add=True)
        pltpu.sync_copy(erows_v, agge_sh.at[idxd_v], add=True)
        return carry

    lax.fori_loop(0, nch, body, 0)
    plsc.subcore_barrier()

    _drain_shared_slice(aggh_sh, rows_v, aggh_out, cid, row0)
    _drain_shared_slice(agge_sh, erows_v, agge_out, cid, row0)


@jax.jit
def _agg_sc(h, edge_flat, edge_feat):
    return pl.kernel(
        _agg_body,
        out_type=[
            jax.ShapeDtypeStruct((NSC, NP, D_FEAT), jnp.float32),
            jax.ShapeDtypeStruct((NSC, NP, D_EDGE), jnp.float32),
        ],
        mesh=_MESH,
        scratch_types=[
            pltpu.VMEM_SHARED((NP, D_FEAT), jnp.float32),
            pltpu.VMEM_SHARED((NP, D_EDGE), jnp.float32),
            pltpu.VMEM((CH,), jnp.int32),
            pltpu.VMEM((CH,), jnp.int32),
            pltpu.VMEM((CH, D_FEAT), jnp.float32),
            pltpu.VMEM((CH, D_EDGE), jnp.float32),
            pltpu.VMEM((64, D_FEAT), jnp.float32),
            pltpu.VMEM((64, D_EDGE), jnp.float32),
            pltpu.SemaphoreType.DMA,
        ],
        compiler_params=_SC_PARAMS,
    )(h, edge_flat, edge_feat)


def _final_body(aggh_ref, agge_ref, deg_ref, we_ref, bias_ref, out_ref):
    s = aggh_ref[0] + aggh_ref[1]
    e = agge_ref[0] + agge_ref[1]
    nd = lax.rsqrt(jnp.maximum(deg_ref[0, :, 0:1], 1.0))
    r = s + jnp.dot(e, we_ref[...], preferred_element_type=jnp.float32)
    out_ref[...] = r * nd + bias_ref[...]


@jax.jit
def _final_tc(aggh, agge, deg, we, bias):
    blk = 1000
    return pl.pallas_call(
        _final_body,
        grid=(N // blk,),
        in_specs=[
            pl.BlockSpec((NSC, blk, D_FEAT), lambda i: (0, i, 0)),
            pl.BlockSpec((NSC, blk, D_EDGE), lambda i: (0, i, 0)),
            pl.BlockSpec((1, blk, 16), lambda i: (1, i, 0)),
            pl.BlockSpec((D_EDGE, D_OUT), lambda i: (0, 0)),
            pl.BlockSpec((1, D_OUT), lambda i: (0, 0)),
        ],
        out_specs=pl.BlockSpec((blk, D_OUT), lambda i: (i, 0)),
        out_shape=jax.ShapeDtypeStruct((N, D_OUT), jnp.float32),
    )(aggh, agge, deg, we, bias[None, :])


def kernel(feat, edge_index, edge_feat, weight, bias):
    e = edge_index.shape[1]
    edge_flat = edge_index.reshape(2 * e)
    wf = weight[:D_FEAT]
    we = weight[D_FEAT:]
    deg = _deg_sc(edge_flat)
    h = _h_tc(deg, feat, wf)
    aggh, agge = _agg_sc(h, edge_flat, edge_feat)
    return _final_tc(aggh, agge, deg, we, bias)


# speedup vs baseline: 8.0121x; 1.4798x over previous
import functools

import jax
import jax.numpy as jnp
from jax import lax
from jax.experimental import pallas as pl
from jax.experimental.pallas import tpu as pltpu
from jax.experimental.pallas import tpu_sc as plsc

N = 10000
D_FEAT = 128
D_EDGE = 16
D_OUT = 128
CH = 128
NP = N + 240
RPT = NP // 16
NSC = 2
NSUB = 16
NW = NSC * NSUB

_MESH = plsc.VectorSubcoreMesh(core_axis_name="c", subcore_axis_name="s")
_SC_PARAMS = pltpu.CompilerParams(use_tc_tiling_on_sc=False)


def _zero_vmem_2d(buf, rows):
    cols = buf.shape[1]

    def body(i, carry):
        for cc in range(cols // 16):
            buf[i, pl.ds(cc * 16, 16)] = jnp.zeros((16,), jnp.float32)
        return carry

    lax.fori_loop(0, rows, body, 0)


def _zero_shared_slice(zbuf, shared, row0):
    for j in range(RPT // 64):
        pltpu.sync_copy(zbuf, shared.at[pl.ds(row0 + j * 64, 64), :])
    rem = RPT % 64
    if rem:
        pltpu.sync_copy(
            zbuf.at[pl.ds(0, rem), :],
            shared.at[pl.ds(row0 + (RPT // 64) * 64, rem), :],
        )


def _drain_shared_slice(shared, stage, out_ref, cid, row0):
    nfull, rem = RPT // CH, RPT % CH
    for j in range(nfull):
        pltpu.sync_copy(shared.at[pl.ds(row0 + j * CH, CH), :], stage)
        pltpu.sync_copy(stage, out_ref.at[cid, pl.ds(row0 + j * CH, CH), :])
    if rem:
        pltpu.sync_copy(
            shared.at[pl.ds(row0 + nfull * CH, rem), :],
            stage.at[pl.ds(0, rem), :],
        )
        pltpu.sync_copy(
            stage.at[pl.ds(0, rem), :],
            out_ref.at[cid, pl.ds(row0 + nfull * CH, rem), :],
        )


def _deg_body(edge_ref, out_ref, deg_sh, ones_v, idx_v, zbuf, stage):
    cid = lax.axis_index("c")
    sid = lax.axis_index("s")
    nedge = edge_ref.shape[0] // 2
    nrow = nedge // CH

    _zero_vmem_2d(zbuf, 64)

    def ones_body(i, carry):
        ones_v[i, :] = jnp.ones((16,), jnp.float32)
        return carry

    lax.fori_loop(0, CH, ones_body, 0)

    row0 = sid * RPT
    _zero_shared_slice(zbuf, deg_sh, row0)
    plsc.subcore_barrier()

    nch = jnp.where(sid < nrow % NSUB, nrow // NSUB + 1, nrow // NSUB)

    def body(k, carry):
        row = sid + NSUB * k
        pltpu.sync_copy(edge_ref.at[pl.ds(cid * nedge + row * CH, CH)], idx_v)
        pltpu.sync_copy(ones_v, deg_sh.at[idx_v], add=True)
        return carry

    lax.fori_loop(0, nch, body, 0)
    plsc.subcore_barrier()

    pltpu.sync_copy(deg_sh.at[pl.ds(row0, RPT), :], stage)
    pltpu.sync_copy(stage, out_ref.at[cid, pl.ds(row0, RPT), :])


@jax.jit
def _deg_sc(edge_flat):
    return pl.kernel(
        _deg_body,
        out_type=jax.ShapeDtypeStruct((NSC, NP, 16), jnp.float32),
        mesh=_MESH,
        scratch_types=[
            pltpu.VMEM_SHARED((NP, 16), jnp.float32),
            pltpu.VMEM((CH, 16), jnp.float32),
            pltpu.VMEM((CH,), jnp.int32),
            pltpu.VMEM((64, 16), jnp.float32),
            pltpu.VMEM((RPT, 16), jnp.float32),
        ],
        compiler_params=_SC_PARAMS,
    )(edge_flat)


def _h_body(deg_ref, feat_ref, wf_ref, h_ref):
    d = deg_ref[0, :, 0:1]
    nd = lax.rsqrt(jnp.maximum(d, 1.0))
    h_ref[...] = jnp.dot(
        feat_ref[...] * nd, wf_ref[...], preferred_element_type=jnp.float32
    )


@jax.jit
def _h_tc(deg, feat, wf):
    blk = 1000
    return pl.pallas_call(
        _h_body,
        grid=(N // blk,),
        in_specs=[
            pl.BlockSpec((1, blk, 16), lambda i: (0, i, 0)),
            pl.BlockSpec((blk, D_FEAT), lambda i: (i, 0)),
            pl.BlockSpec((D_FEAT, D_OUT), lambda i: (0, 0)),
        ],
        out_specs=pl.BlockSpec((blk, D_OUT), lambda i: (i, 0)),
        out_shape=jax.ShapeDtypeStruct((N, D_OUT), jnp.float32),
    )(deg, feat, wf)


SLOT_E = CH
SLOTS_PW = 78
LEFT_ROWS = 4


def _agg_body(h_ref, edge_ref, ef_ref, zh_ref, ze_ref, aggh_out, agge_out,
              aggh_sh, agge_sh, idxs_v, idxd_v, rows_v, erows_v,
              seml0, seml1, semg0, semg1, sems0, sems1):
    cid = lax.axis_index("c")
    sid = lax.axis_index("s")
    wid = cid * NSUB + sid
    nedge = edge_ref.shape[0] // 2
    base = wid * SLOTS_PW * SLOT_E

    seml = (seml0, seml1)
    semg = (semg0, semg1)
    sems = (sems0, sems1)

    row0 = sid * RPT
    pltpu.sync_copy(zh_ref, aggh_sh.at[pl.ds(row0, RPT), :])
    pltpu.sync_copy(ze_ref, agge_sh.at[pl.ds(row0, RPT), :])
    plsc.subcore_barrier()

    def issue_load(j, s):
        eoff = base + j * SLOT_E
        pltpu.async_copy(edge_ref.at[pl.ds(eoff, SLOT_E)], idxs_v.at[s], seml[s])
        pltpu.async_copy(edge_ref.at[pl.ds(nedge + eoff, SLOT_E)],
                         idxd_v.at[s, 0], seml[s])
        pltpu.async_copy(ef_ref.at[pl.ds(eoff, SLOT_E), :], erows_v.at[s], seml[s])

    def wait_load(s):
        pltpu.make_async_copy(edge_ref.at[pl.ds(0, SLOT_E)], idxs_v.at[s], seml[s]).wait()
        pltpu.make_async_copy(edge_ref.at[pl.ds(0, SLOT_E)], idxd_v.at[s, 0], seml[s]).wait()
        pltpu.make_async_copy(ef_ref.at[pl.ds(0, SLOT_E), :], erows_v.at[s], seml[s]).wait()

    def issue_gather(s):
        pltpu.async_copy(h_ref.at[idxs_v.at[s]], rows_v.at[s], semg[s])

    def wait_gather(s):
        pltpu.make_async_copy(h_ref.at[idxs_v.at[s]], rows_v.at[s], semg[s]).wait()

    def issue_scatter(s):
        pltpu.async_copy(rows_v.at[s], aggh_sh.at[idxd_v.at[s, 0]], sems[s], add=True)
        pltpu.async_copy(erows_v.at[s], agge_sh.at[idxd_v.at[s, 0]], sems[s], add=True)

    def wait_scatter(s):
        pltpu.make_async_copy(rows_v.at[s], aggh_sh.at[idxd_v.at[s, 0]], sems[s]).wait()
        pltpu.make_async_copy(erows_v.at[s], agge_sh.at[idxd_v.at[s, 0]], sems[s]).wait()

    issue_load(0, 0)
    issue_load(1, 1)
    wait_load(0)
    issue_gather(0)
    last_it = SLOTS_PW // 2 - 1

    def body(it, carry):
        wait_gather(0)
        issue_scatter(0)
        wait_load(1)
        issue_gather(1)
        wait_scatter(0)

        @pl.when(it < last_it)
        def _():
            issue_load(2 * it + 2, 0)

        wait_gather(1)
        issue_scatter(1)

        @pl.when(it < last_it)
        def _():
            wait_load(0)
            issue_gather(0)

        wait_scatter(1)

        @pl.when(it < last_it)
        def _():
            issue_load(2 * it + 3, 1)

        return carry

    lax.fori_loop(0, SLOTS_PW // 2, body, 0)

    @pl.when(wid < LEFT_ROWS)
    def _():
        eoff = SLOTS_PW * SLOT_E * NW + wid * CH
        pltpu.sync_copy(edge_ref.at[pl.ds(eoff, CH)], idxs_v.at[0])
        pltpu.sync_copy(edge_ref.at[pl.ds(nedge + eoff, CH)], idxd_v.at[0, 0])
        pltpu.sync_copy(ef_ref.at[pl.ds(eoff, CH), :], erows_v.at[0])
        pltpu.async_copy(h_ref.at[idxs_v.at[0]], rows_v.at[0], semg0).wait()
        pltpu.sync_copy(rows_v.at[0], aggh_sh.at[idxd_v.at[0, 0]], add=True)
        pltpu.sync_copy(erows_v.at[0], agge_sh.at[idxd_v.at[0, 0]], add=True)

    plsc.subcore_barrier()

    _drain_shared_slice(aggh_sh, rows_v.at[0], aggh_out, cid, row0)
    _drain_shared_slice(agge_sh, erows_v.at[0], agge_out, cid, row0)


@jax.jit
def _agg_sc(h, edge_flat, edge_feat, zh, ze):
    return pl.kernel(
        _agg_body,
        out_type=[
            jax.ShapeDtypeStruct((NSC, NP, D_FEAT), jnp.float32),
            jax.ShapeDtypeStruct((NSC, NP, D_EDGE), jnp.float32),
        ],
        mesh=_MESH,
        scratch_types=[
            pltpu.VMEM_SHARED((NP, D_FEAT), jnp.float32),
            pltpu.VMEM_SHARED((NP, D_EDGE), jnp.float32),
            pltpu.VMEM((2, SLOT_E), jnp.int32),
            pltpu.VMEM((2, 1, CH), jnp.int32),
            pltpu.VMEM((2, SLOT_E, D_FEAT), jnp.float32),
            pltpu.VMEM((2, SLOT_E, D_EDGE), jnp.float32),
            pltpu.SemaphoreType.DMA,
            pltpu.SemaphoreType.DMA,
            pltpu.SemaphoreType.DMA,
            pltpu.SemaphoreType.DMA,
            pltpu.SemaphoreType.DMA,
            pltpu.SemaphoreType.DMA,
        ],
        compiler_params=_SC_PARAMS,
    )(h, edge_flat, edge_feat, zh, ze)


def _final_body(aggh_ref, agge_ref, deg_ref, we_ref, bias_ref, out_ref):
    s = aggh_ref[0] + aggh_ref[1]
    e = agge_ref[0] + agge_ref[1]
    nd = lax.rsqrt(jnp.maximum(deg_ref[0, :, 0:1], 1.0))
    r = s + jnp.dot(e, we_ref[...], preferred_element_type=jnp.float32)
    out_ref[...] = r * nd + bias_ref[...]


@jax.jit
def _final_tc(aggh, agge, deg, we, bias):
    blk = 1000
    return pl.pallas_call(
        _final_body,
        grid=(N // blk,),
        in_specs=[
            pl.BlockSpec((NSC, blk, D_FEAT), lambda i: (0, i, 0)),
            pl.BlockSpec((NSC, blk, D_EDGE), lambda i: (0, i, 0)),
            pl.BlockSpec((1, blk, 16), lambda i: (1, i, 0)),
            pl.BlockSpec((D_EDGE, D_OUT), lambda i: (0, 0)),
            pl.BlockSpec((1, D_OUT), lambda i: (0, 0)),
        ],
        out_specs=pl.BlockSpec((blk, D_OUT), lambda i: (i, 0)),
        out_shape=jax.ShapeDtypeStruct((N, D_OUT), jnp.float32),
    )(aggh, agge, deg, we, bias[None, :])


def kernel(feat, edge_index, edge_feat, weight, bias):
    e = edge_index.shape[1]
    edge_flat = edge_index.reshape(2 * e)
    wf = weight[:D_FEAT]
    we = weight[D_FEAT:]
    zh = jnp.zeros((RPT, D_FEAT), jnp.float32)
    ze = jnp.zeros((RPT, D_EDGE), jnp.float32)
    deg = _deg_sc(edge_flat)
    h = _h_tc(deg, feat, wf)
    aggh, agge = _agg_sc(h, edge_flat, edge_feat, zh, ze)
    return _final_tc(aggh, agge, deg, we, bias)


# speedup vs baseline: 8.4887x; 1.0595x over previous
import functools

import jax
import jax.numpy as jnp
from jax import lax
from jax.experimental import pallas as pl
from jax.experimental.pallas import tpu as pltpu
from jax.experimental.pallas import tpu_sc as plsc

N = 10000
D_FEAT = 128
D_EDGE = 16
D_OUT = 128
CH = 128
NP = N + 240
RPT = NP // 16
NSC = 2
NSUB = 16
NW = NSC * NSUB

_MESH = plsc.VectorSubcoreMesh(core_axis_name="c", subcore_axis_name="s")
_SC_PARAMS = pltpu.CompilerParams(use_tc_tiling_on_sc=False)
_SC_PARAMS_NL = pltpu.CompilerParams(
    use_tc_tiling_on_sc=False, needs_layout_passes=False)


def _drain_shared_slice(shared, stage, out_ref, cid, row0):
    nfull, rem = RPT // CH, RPT % CH
    for j in range(nfull):
        pltpu.sync_copy(shared.at[pl.ds(row0 + j * CH, CH), :], stage)
        pltpu.sync_copy(stage, out_ref.at[cid, pl.ds(row0 + j * CH, CH), :])
    if rem:
        pltpu.sync_copy(
            shared.at[pl.ds(row0 + nfull * CH, rem), :],
            stage.at[pl.ds(0, rem), :],
        )
        pltpu.sync_copy(
            stage.at[pl.ds(0, rem), :],
            out_ref.at[cid, pl.ds(row0 + nfull * CH, rem), :],
        )


IDXCH = 2000


def _deg_body(edge_ref, out_ref, stage_sh, cnt_v, ibuf_v, red_v, st2_v,
              seml0, seml1):
    cid = lax.axis_index("c")
    sid = lax.axis_index("s")
    nedge = edge_ref.shape[0] // 2
    ept = nedge // NSUB
    base = cid * nedge + sid * ept
    seml = (seml0, seml1)

    def zero(i, carry):
        cnt_v[pl.ds(i * 16, 16)] = jnp.zeros((16,), jnp.float32)
        return carry

    lax.fori_loop(0, NP // 16, zero, 0)

    nchunk = ept // IDXCH

    def issue(k, s):
        pltpu.async_copy(edge_ref.at[pl.ds(base + k * IDXCH, IDXCH)],
                         ibuf_v.at[s], seml[s])

    def wait(s):
        pltpu.make_async_copy(edge_ref.at[pl.ds(0, IDXCH)],
                              ibuf_v.at[s], seml[s]).wait()

    issue(0, 0)
    for k in range(nchunk):
        s = k % 2
        wait(s)
        if k + 1 < nchunk:
            issue(k + 1, 1 - s)

        def group(g, carry):
            for u in range(5):
                idx = ibuf_v[s, pl.ds((g * 5 + u) * 16, 16)]
                cnts, last = plsc.scan_count(idx)
                plsc.addupdate_scatter(
                    cnt_v, [idx], cnts.astype(jnp.float32), mask=last)
            return carry

        lax.fori_loop(0, IDXCH // 80, group, 0)

    pltpu.sync_copy(cnt_v, stage_sh.at[sid])
    plsc.subcore_barrier()

    for t in range(NSUB):
        pltpu.sync_copy(stage_sh.at[t, pl.ds(sid * RPT, RPT)], red_v.at[t])

    lanes = lax.iota(jnp.int32, 16)
    zeros16 = jnp.zeros((16,), jnp.int32)

    def red(g, carry):
        acc = red_v[0, pl.ds(g * 16, 16)]
        for t in range(1, NSUB):
            acc = acc + red_v[t, pl.ds(g * 16, 16)]
        plsc.store_scatter(st2_v, [g * 16 + lanes, zeros16], acc)
        return carry

    lax.fori_loop(0, RPT // 16, red, 0)
    pltpu.sync_copy(st2_v, out_ref.at[cid, pl.ds(sid * RPT, RPT), :])


@jax.jit
def _deg_sc(edge_flat):
    return pl.kernel(
        _deg_body,
        out_type=jax.ShapeDtypeStruct((NSC, NP, 16), jnp.float32),
        mesh=_MESH,
        scratch_types=[
            pltpu.VMEM_SHARED((NSUB, NP), jnp.float32),
            pltpu.VMEM((NP,), jnp.float32),
            pltpu.VMEM((2, IDXCH), jnp.int32),
            pltpu.VMEM((NSUB, RPT), jnp.float32),
            pltpu.VMEM((RPT, 16), jnp.float32),
            pltpu.SemaphoreType.DMA,
            pltpu.SemaphoreType.DMA,
        ],
        compiler_params=_SC_PARAMS_NL,
    )(edge_flat)


def _h_body(deg_ref, feat_ref, wf_ref, h_ref):
    d = deg_ref[0, :, 0:1]
    nd = lax.rsqrt(jnp.maximum(d, 1.0))
    h_ref[...] = jnp.dot(
        feat_ref[...] * nd, wf_ref[...], preferred_element_type=jnp.float32
    )


@jax.jit
def _h_tc(deg, feat, wf):
    blk = 1000
    return pl.pallas_call(
        _h_body,
        grid=(N // blk,),
        in_specs=[
            pl.BlockSpec((1, blk, 16), lambda i: (0, i, 0)),
            pl.BlockSpec((blk, D_FEAT), lambda i: (i, 0)),
            pl.BlockSpec((D_FEAT, D_OUT), lambda i: (0, 0)),
        ],
        out_specs=pl.BlockSpec((blk, D_OUT), lambda i: (i, 0)),
        out_shape=jax.ShapeDtypeStruct((N, D_OUT), jnp.float32),
    )(deg, feat, wf)


SLOT_E = CH
SLOTS_PW = 78
LEFT_ROWS = 4


def _agg_body(h_ref, edge_ref, ef_ref, zh_ref, ze_ref, aggh_out, agge_out,
              aggh_sh, agge_sh, idxs_v, idxd_v, rows_v, erows_v,
              seml0, seml1, semg0, semg1, sems0, sems1):
    cid = lax.axis_index("c")
    sid = lax.axis_index("s")
    wid = cid * NSUB + sid
    nedge = edge_ref.shape[0] // 2
    base = wid * SLOTS_PW * SLOT_E

    seml = (seml0, seml1)
    semg = (semg0, semg1)
    sems = (sems0, sems1)

    row0 = sid * RPT
    pltpu.sync_copy(zh_ref, aggh_sh.at[pl.ds(row0, RPT), :])
    pltpu.sync_copy(ze_ref, agge_sh.at[pl.ds(row0, RPT), :])
    plsc.subcore_barrier()

    def issue_load(j, s):
        eoff = base + j * SLOT_E
        pltpu.async_copy(edge_ref.at[pl.ds(eoff, SLOT_E)], idxs_v.at[s], seml[s])
        pltpu.async_copy(edge_ref.at[pl.ds(nedge + eoff, SLOT_E)],
                         idxd_v.at[s, 0], seml[s])
        pltpu.async_copy(ef_ref.at[pl.ds(eoff, SLOT_E), :], erows_v.at[s], seml[s])

    def wait_load(s):
        pltpu.make_async_copy(edge_ref.at[pl.ds(0, SLOT_E)], idxs_v.at[s], seml[s]).wait()
        pltpu.make_async_copy(edge_ref.at[pl.ds(0, SLOT_E)], idxd_v.at[s, 0], seml[s]).wait()
        pltpu.make_async_copy(ef_ref.at[pl.ds(0, SLOT_E), :], erows_v.at[s], seml[s]).wait()

    def issue_gather(s):
        pltpu.async_copy(h_ref.at[idxs_v.at[s]], rows_v.at[s], semg[s])

    def wait_gather(s):
        pltpu.make_async_copy(h_ref.at[idxs_v.at[s]], rows_v.at[s], semg[s]).wait()

    def issue_scatter(s):
        pltpu.async_copy(rows_v.at[s], aggh_sh.at[idxd_v.at[s, 0]], sems[s], add=True)
        pltpu.async_copy(erows_v.at[s], agge_sh.at[idxd_v.at[s, 0]], sems[s], add=True)

    def wait_scatter(s):
        pltpu.make_async_copy(rows_v.at[s], aggh_sh.at[idxd_v.at[s, 0]], sems[s]).wait()
        pltpu.make_async_copy(erows_v.at[s], agge_sh.at[idxd_v.at[s, 0]], sems[s]).wait()

    issue_load(0, 0)
    issue_load(1, 1)
    wait_load(0)
    issue_gather(0)
    last_it = SLOTS_PW // 2 - 1

    def body(it, carry):
        wait_gather(0)
        issue_scatter(0)
        wait_load(1)
        issue_gather(1)
        wait_scatter(0)

        @pl.when(it < last_it)
        def _():
            issue_load(2 * it + 2, 0)

        wait_gather(1)
        issue_scatter(1)

        @pl.when(it < last_it)
        def _():
            wait_load(0)
            issue_gather(0)

        wait_scatter(1)

        @pl.when(it < last_it)
        def _():
            issue_load(2 * it + 3, 1)

        return carry

    lax.fori_loop(0, SLOTS_PW // 2, body, 0)

    @pl.when(wid < LEFT_ROWS)
    def _():
        eoff = SLOTS_PW * SLOT_E * NW + wid * CH
        pltpu.sync_copy(edge_ref.at[pl.ds(eoff, CH)], idxs_v.at[0])
        pltpu.sync_copy(edge_ref.at[pl.ds(nedge + eoff, CH)], idxd_v.at[0, 0])
        pltpu.sync_copy(ef_ref.at[pl.ds(eoff, CH), :], erows_v.at[0])
        pltpu.async_copy(h_ref.at[idxs_v.at[0]], rows_v.at[0], semg0).wait()
        pltpu.sync_copy(rows_v.at[0], aggh_sh.at[idxd_v.at[0, 0]], add=True)
        pltpu.sync_copy(erows_v.at[0], agge_sh.at[idxd_v.at[0, 0]], add=True)

    plsc.subcore_barrier()

    _drain_shared_slice(aggh_sh, rows_v.at[0], aggh_out, cid, row0)
    _drain_shared_slice(agge_sh, erows_v.at[0], agge_out, cid, row0)


@jax.jit
def _agg_sc(h, edge_flat, edge_feat, zh, ze):
    return pl.kernel(
        _agg_body,
        out_type=[
            jax.ShapeDtypeStruct((NSC, NP, D_FEAT), jnp.float32),
            jax.ShapeDtypeStruct((NSC, NP, D_EDGE), jnp.float32),
        ],
        mesh=_MESH,
        scratch_types=[
            pltpu.VMEM_SHARED((NP, D_FEAT), jnp.float32),
            pltpu.VMEM_SHARED((NP, D_EDGE), jnp.float32),
            pltpu.VMEM((2, SLOT_E), jnp.int32),
            pltpu.VMEM((2, 1, CH), jnp.int32),
            pltpu.VMEM((2, SLOT_E, D_FEAT), jnp.float32),
            pltpu.VMEM((2, SLOT_E, D_EDGE), jnp.float32),
            pltpu.SemaphoreType.DMA,
            pltpu.SemaphoreType.DMA,
            pltpu.SemaphoreType.DMA,
            pltpu.SemaphoreType.DMA,
            pltpu.SemaphoreType.DMA,
            pltpu.SemaphoreType.DMA,
        ],
        compiler_params=_SC_PARAMS,
    )(h, edge_flat, edge_feat, zh, ze)


def _final_body(aggh_ref, agge_ref, deg_ref, we_ref, bias_ref, out_ref):
    s = aggh_ref[0] + aggh_ref[1]
    e = agge_ref[0] + agge_ref[1]
    nd = lax.rsqrt(jnp.maximum(deg_ref[0, :, 0:1], 1.0))
    r = s + jnp.dot(e, we_ref[...], preferred_element_type=jnp.float32)
    out_ref[...] = r * nd + bias_ref[...]


@jax.jit
def _final_tc(aggh, agge, deg, we, bias):
    blk = 1000
    return pl.pallas_call(
        _final_body,
        grid=(N // blk,),
        in_specs=[
            pl.BlockSpec((NSC, blk, D_FEAT), lambda i: (0, i, 0)),
            pl.BlockSpec((NSC, blk, D_EDGE), lambda i: (0, i, 0)),
            pl.BlockSpec((1, blk, 16), lambda i: (1, i, 0)),
            pl.BlockSpec((D_EDGE, D_OUT), lambda i: (0, 0)),
            pl.BlockSpec((1, D_OUT), lambda i: (0, 0)),
        ],
        out_specs=pl.BlockSpec((blk, D_OUT), lambda i: (i, 0)),
        out_shape=jax.ShapeDtypeStruct((N, D_OUT), jnp.float32),
    )(aggh, agge, deg, we, bias[None, :])


def kernel(feat, edge_index, edge_feat, weight, bias):
    e = edge_index.shape[1]
    edge_flat = edge_index.reshape(2 * e)
    wf = weight[:D_FEAT]
    we = weight[D_FEAT:]
    zh = jnp.zeros((RPT, D_FEAT), jnp.float32)
    ze = jnp.zeros((RPT, D_EDGE), jnp.float32)
    deg = _deg_sc(edge_flat)
    h = _h_tc(deg, feat, wf)
    aggh, agge = _agg_sc(h, edge_flat, edge_feat, zh, ze)
    return _final_tc(aggh, agge, deg, we, bias)
